# baseline (speedup 1.0000x reference)
import functools

import jax
import jax.numpy as jnp
from jax import lax
from jax.experimental import pallas as pl
from jax.experimental.pallas import tpu as pltpu
from jax.experimental.pallas import tpu_sc as plsc

_NUM_ROWS = 1000000
_DIM = 64
_CHUNK = 200
_NCHUNKS = _NUM_ROWS // _CHUNK
_NW = 32
_PER_W = _NCHUNKS // _NW
_TAIL = _NCHUNKS - _PER_W * _NW
_NBUF = 4
_LOOKAHEAD = 2


def _sc_copy(src_hbm, out_hbm, buf, in_sems, out_sems):
    c = lax.axis_index("c")
    s = lax.axis_index("s")
    wid = s * 2 + c

    def in_cp(i):
        base = (i * _NW + wid) * _CHUNK
        return pltpu.make_async_copy(
            src_hbm.at[pl.ds(base, _CHUNK)], buf.at[i % _NBUF],
            in_sems.at[i % _NBUF])

    def out_cp(i):
        base = (i * _NW + wid) * _CHUNK
        return pltpu.make_async_copy(
            buf.at[i % _NBUF], out_hbm.at[pl.ds(base, _CHUNK)],
            out_sems.at[i % _NBUF])

    for i in range(_LOOKAHEAD):
        in_cp(i).start()
    for i in range(_PER_W):
        in_cp(i).wait()
        out_cp(i).start()
        nxt = i + _LOOKAHEAD
        if nxt < _PER_W:
            if nxt >= _NBUF:
                out_cp(nxt - _NBUF).wait()
            in_cp(nxt).start()
    for i in range(max(0, _PER_W - _NBUF), _PER_W):
        out_cp(i).wait()

    @pl.when(wid < _TAIL)
    def _():
        base = (_PER_W * _NW + wid) * _CHUNK
        pltpu.sync_copy(src_hbm.at[pl.ds(base, _CHUNK)], buf.at[0])
        pltpu.sync_copy(buf.at[0], out_hbm.at[pl.ds(base, _CHUNK)])


def kernel(embeddings):
    mesh = plsc.VectorSubcoreMesh(core_axis_name="c", subcore_axis_name="s")
    k = functools.partial(
        pl.kernel,
        mesh=mesh,
        out_type=jax.ShapeDtypeStruct(embeddings.shape, embeddings.dtype),
        scratch_types=[
            pltpu.VMEM((_NBUF, _CHUNK, _DIM), embeddings.dtype),
            pltpu.SemaphoreType.DMA((_NBUF,)),
            pltpu.SemaphoreType.DMA((_NBUF,)),
        ],
    )(_sc_copy)
    return k(embeddings)


# speedup vs baseline: 1.3878x; 1.3878x over previous
import jax
import jax.numpy as jnp
from jax.experimental import pallas as pl
from jax.experimental.pallas import tpu as pltpu

_BR = 1000


def _copy_body(src_ref, dst_ref):
    dst_ref[...] = src_ref[...]


def kernel(embeddings):
    rows, dim = embeddings.shape
    v = embeddings.reshape(8, rows // 8, dim)
    grid = (rows // 8) // _BR
    out = pl.pallas_call(
        _copy_body,
        out_shape=jax.ShapeDtypeStruct(v.shape, v.dtype),
        grid=(grid,),
        in_specs=[pl.BlockSpec((8, _BR, dim), lambda i: (0, i, 0))],
        out_specs=pl.BlockSpec((8, _BR, dim), lambda i: (0, i, 0)),
    )(v)
    return out.reshape(rows, dim)
